# baseline (speedup 1.0000x reference)
import jax
import jax.numpy as jnp
from jax import lax
from jax.experimental import pallas as pl
from jax.experimental.pallas import tpu as pltpu
from jax.experimental.pallas import tpu_sc as plsc

B, S, D = 16384, 100, 64
K = 26
NC, NS, L = 2, 16, 16
NW = NC * NS
TPK = D // 8


def _body(x_ref, idx_ref, out_ref, idx_v, sem):
    wid = lax.axis_index("s") * NC + lax.axis_index("c")
    pltpu.sync_copy(idx_ref, idx_v)
    ga = idx_v[pl.ds(0, L)]
    gb = idx_v[pl.ds(K - L, L)]
    idx = [ga[k] for k in range(L)] + [gb[k - (K - L)] for k in range(L, K)]

    def task_refs(k, tr):
        src0 = pl.multiple_of(idx[k] * D, 8)
        return (x_ref.at[pl.ds(src0 + tr * 8, 8)],
                out_ref.at[pl.ds(k * D + tr * 8, 8)])

    for k in range(K):
        for tr in range(TPK):
            @pl.when(wid == (k * TPK + tr) % NW)
            def _(k=k, tr=tr):
                src, dst = task_refs(k, tr)
                pltpu.async_copy(src, dst, sem)
    for k in range(K):
        for tr in range(TPK):
            @pl.when(wid == (k * TPK + tr) % NW)
            def _(k=k, tr=tr):
                src, dst = task_refs(k, tr)
                pltpu.make_async_copy(src, dst, sem).wait()


def _slab_copy(x2, indices):
    mesh = plsc.VectorSubcoreMesh(core_axis_name="c", subcore_axis_name="s",
                                  num_cores=NC, num_subcores=NS)
    return pl.kernel(
        _body,
        out_type=jax.ShapeDtypeStruct((K * D, B), jnp.float32),
        mesh=mesh,
        scratch_types=[
            pltpu.VMEM((K,), jnp.int32),
            pltpu.SemaphoreType.DMA,
        ],
    )(x2, indices)


def kernel(x, indices):
    x2 = x.transpose(1, 2, 0).reshape(S * D, B)
    out2 = _slab_copy(x2, indices)
    return out2.reshape(K, D, B).transpose(2, 0, 1)


# speedup vs baseline: 34.6488x; 34.6488x over previous
import jax
import jax.numpy as jnp
from jax import lax
from jax.experimental import pallas as pl
from jax.experimental.pallas import tpu as pltpu
from jax.experimental.pallas import tpu_sc as plsc

B, S, D = 16384, 100, 64
K = 26
NC, NS, L = 2, 16, 16
NW = NC * NS
CW = B // NW
NBUF = 3
LEAD = 2


def _body(x_ref, idx_ref, out_ref, idx_v, bv0, bv1, bv2, g0, g1, g2,
          s0, s1, s2):
    bufs = (bv0, bv1, bv2)
    gsem = (g0, g1, g2)
    ssem = (s0, s1, s2)
    wid = lax.axis_index("s") * NC + lax.axis_index("c")
    c0 = wid * CW
    pltpu.sync_copy(idx_ref, idx_v)
    ga = idx_v[pl.ds(0, L)]
    gb = idx_v[pl.ds(K - L, L)]
    idx = [ga[k] for k in range(L)] + [gb[k - (K - L)] for k in range(L, K)]

    def src(k):
        return x_ref.at[pl.ds(pl.multiple_of(idx[k] * D, 8), D),
                        pl.ds(c0, CW)]

    def dst(k):
        return out_ref.at[pl.ds(k * D, D), pl.ds(c0, CW)]

    for k in range(LEAD):
        pltpu.async_copy(src(k), bufs[k % NBUF], gsem[k % NBUF])
    for k in range(K):
        slot = k % NBUF
        nxt = k + LEAD
        if nxt < K:
            nslot = nxt % NBUF
            if nxt - NBUF >= 0:
                pltpu.make_async_copy(bufs[nslot], dst(nxt - NBUF),
                                      ssem[nslot]).wait()
            pltpu.async_copy(src(nxt), bufs[nslot], gsem[nslot])
        pltpu.make_async_copy(src(k), bufs[slot], gsem[slot]).wait()
        pltpu.async_copy(bufs[slot], dst(k), ssem[slot])
    for k in range(K - NBUF, K):
        slot = k % NBUF
        pltpu.make_async_copy(bufs[slot], dst(k), ssem[slot]).wait()


def _slab_copy(x2, indices):
    mesh = plsc.VectorSubcoreMesh(core_axis_name="c", subcore_axis_name="s",
                                  num_cores=NC, num_subcores=NS)
    return pl.kernel(
        _body,
        out_type=jax.ShapeDtypeStruct((K * D, B), jnp.float32),
        mesh=mesh,
        scratch_types=[
            pltpu.VMEM((K,), jnp.int32),
        ] + [pltpu.VMEM((D, CW), jnp.float32) for _ in range(NBUF)]
          + [pltpu.SemaphoreType.DMA for _ in range(2 * NBUF)],
    )(x2, indices)


def kernel(x, indices):
    x2 = x.transpose(1, 2, 0).reshape(S * D, B)
    out2 = _slab_copy(x2, indices)
    return out2.reshape(K, D, B).transpose(2, 0, 1)


# speedup vs baseline: 34.7063x; 1.0017x over previous
import jax
import jax.numpy as jnp
from jax import lax
from jax.experimental import pallas as pl
from jax.experimental.pallas import tpu as pltpu
from jax.experimental.pallas import tpu_sc as plsc

B, S, D = 16384, 100, 64
K = 26
NC, NS, L = 2, 16, 16
NW = NC * NS
QC = B // 4
NBUF = 3
LEAD = 2


def _body(x_ref, idx_ref, out_ref, idx_v, bv0, bv1, bv2, g0, g1, g2,
          s0, s1, s2):
    bufs = (bv0, bv1, bv2)
    gsem = (g0, g1, g2)
    ssem = (s0, s1, s2)
    c_id = lax.axis_index("c")
    s_id = lax.axis_index("s")
    q = (c_id << 1) | (s_id >> 3)
    r = s_id & 7
    col0 = q * QC
    row8 = r * 8
    pltpu.sync_copy(idx_ref, idx_v)
    ga = idx_v[pl.ds(0, L)]
    gb = idx_v[pl.ds(K - L, L)]
    idx = [ga[k] for k in range(L)] + [gb[k - (K - L)] for k in range(L, K)]

    def src(k):
        return x_ref.at[pl.ds(pl.multiple_of(idx[k] * D, 8) + row8, 8),
                        pl.ds(col0, QC)]

    def dst(k):
        return out_ref.at[pl.ds(k * D + row8, 8), pl.ds(col0, QC)]

    for k in range(LEAD):
        pltpu.async_copy(src(k), bufs[k % NBUF], gsem[k % NBUF])
    for k in range(K):
        slot = k % NBUF
        nxt = k + LEAD
        if nxt < K:
            nslot = nxt % NBUF
            if nxt - NBUF >= 0:
                pltpu.make_async_copy(bufs[nslot], dst(nxt - NBUF),
                                      ssem[nslot]).wait()
            pltpu.async_copy(src(nxt), bufs[nslot], gsem[nslot])
        pltpu.make_async_copy(src(k), bufs[slot], gsem[slot]).wait()
        pltpu.async_copy(bufs[slot], dst(k), ssem[slot])
    for k in range(K - NBUF, K):
        slot = k % NBUF
        pltpu.make_async_copy(bufs[slot], dst(k), ssem[slot]).wait()


def _slab_copy(x2, indices):
    mesh = plsc.VectorSubcoreMesh(core_axis_name="c", subcore_axis_name="s",
                                  num_cores=NC, num_subcores=NS)
    return pl.kernel(
        _body,
        out_type=jax.ShapeDtypeStruct((K * D, B), jnp.float32),
        mesh=mesh,
        scratch_types=[
            pltpu.VMEM((K,), jnp.int32),
        ] + [pltpu.VMEM((8, QC), jnp.float32) for _ in range(NBUF)]
          + [pltpu.SemaphoreType.DMA for _ in range(2 * NBUF)],
    )(x2, indices)


def kernel(x, indices):
    x2 = x.transpose(1, 2, 0).reshape(S * D, B)
    out2 = _slab_copy(x2, indices)
    return out2.reshape(K, D, B).transpose(2, 0, 1)
